# Initial kernel scaffold; baseline (speedup 1.0000x reference)
import jax
import jax.numpy as jnp
from jax.experimental import pallas as pl


def kernel(x, edge_index, W_ge, b_ge, emb, W_gin, bn_g, bn_b, W_key, b_key, W_q, W_val, b_val, W_p0, b_p0, W_p1, b_p1):
    raise NotImplementedError("write your pallas kernel here")


# speedup vs baseline: 7.6736x; 7.6736x over previous
import functools

import jax
import jax.numpy as jnp
import numpy as np
from jax import lax
from jax.experimental import pallas as pl
from jax.experimental.pallas import tpu as pltpu
from jax.experimental.pallas import tpu_sc as plsc

B = 2
N_GENE = 19836
D_IN = 11
D_H = 64
N = B * N_GENE
E = 634752

NPAD = 39680
BLK = 2480
GRID = NPAD // BLK
ROWS_PER_TILE = NPAD // 16
E_PAD = E + 128
CHUNK = 128
NCHUNK = E_PAD // (16 * CHUNK)
HALF = D_H // 2

_SQRT1_2 = np.float32(0.7071067811865476)


def _gelu(t):
    return 0.5 * t * (1.0 + lax.erf(t * _SQRT1_2))


def _enc_body(x_ref, emb_ref, wge_ref, bge_ref, lo_ref, hi_ref):
    h = lax.dot_general(x_ref[...], wge_ref[...],
                        (((1,), (1,)), ((), ())),
                        preferred_element_type=jnp.float32,
                        precision=lax.Precision.HIGHEST)
    h = _gelu(h + bge_ref[...]) + emb_ref[...]
    lo_ref[...] = h[:, :HALF]
    hi_ref[...] = h[:, HALF:]


def _encode(xf, emb2, W_ge, b_ge):
    return pl.pallas_call(
        _enc_body,
        grid=(GRID,),
        in_specs=[
            pl.BlockSpec((BLK, D_IN), lambda i: (i, 0)),
            pl.BlockSpec((BLK, D_H), lambda i: (i, 0)),
            pl.BlockSpec((D_H, D_IN), lambda i: (0, 0)),
            pl.BlockSpec((1, D_H), lambda i: (0, 0)),
        ],
        out_specs=[
            pl.BlockSpec((BLK, HALF), lambda i: (i, 0)),
            pl.BlockSpec((BLK, HALF), lambda i: (i, 0)),
        ],
        out_shape=[
            jax.ShapeDtypeStruct((NPAD, HALF), jnp.float32),
            jax.ShapeDtypeStruct((NPAD, HALF), jnp.float32),
        ],
    )(xf, emb2, W_ge, b_ge.reshape(1, D_H))


STAGE = 62
NSTAGE = NCHUNK // STAGE


def _sc_body(hlo, hhi, srcp, dstp, olo, ohi,
             acc, isrc, idst, gb0, gb1, sem0, sem1):
    c = lax.axis_index("c")
    s = lax.axis_index("s")

    def run(h_hbm, out_hbm):
        rows = pl.ds(s * ROWS_PER_TILE, ROWS_PER_TILE)
        pltpu.sync_copy(h_hbm.at[rows], acc.at[rows])
        plsc.subcore_barrier()

        def outer(t, carry0):
            pltpu.sync_copy(srcp.at[s, pl.ds(t * STAGE, STAGE)], isrc)
            pltpu.sync_copy(dstp.at[s, pl.ds(t * STAGE, STAGE)], idst)
            pltpu.async_copy(h_hbm.at[isrc.at[0]], gb0, sem0)

            def step(k, carry):
                i0 = 2 * k
                pltpu.async_copy(h_hbm.at[isrc.at[i0 + 1]], gb1, sem1)
                pltpu.make_async_copy(h_hbm.at[pl.ds(0, CHUNK)], gb0,
                                      sem0).wait()
                pltpu.sync_copy(gb0, acc.at[idst.at[i0]], add=True)

                @pl.when(k + 1 < STAGE // 2)
                def _():
                    pltpu.async_copy(h_hbm.at[isrc.at[i0 + 2]], gb0, sem0)

                pltpu.make_async_copy(h_hbm.at[pl.ds(0, CHUNK)], gb1,
                                      sem1).wait()
                pltpu.sync_copy(gb1, acc.at[idst.at[i0 + 1]], add=True)
                return carry

            lax.fori_loop(0, STAGE // 2, step, 0)
            return carry0

        lax.fori_loop(0, NSTAGE, outer, 0)
        plsc.subcore_barrier()
        pltpu.sync_copy(acc.at[rows], out_hbm.at[rows])

    @pl.when(c == 0)
    def _():
        run(hlo, olo)

    @pl.when(c == 1)
    def _():
        run(hhi, ohi)


@functools.cache
def _build_sc_agg():
    return pl.kernel(
        _sc_body,
        out_type=(
            jax.ShapeDtypeStruct((NPAD, HALF), jnp.float32),
            jax.ShapeDtypeStruct((NPAD, HALF), jnp.float32),
        ),
        mesh=plsc.VectorSubcoreMesh(core_axis_name="c", subcore_axis_name="s"),
        scratch_types=[
            pltpu.VMEM_SHARED((NPAD, HALF), jnp.float32),
            pltpu.VMEM((STAGE, CHUNK), jnp.int32),
            pltpu.VMEM((STAGE, CHUNK), jnp.int32),
            pltpu.VMEM((CHUNK, HALF), jnp.float32),
            pltpu.VMEM((CHUNK, HALF), jnp.float32),
            pltpu.SemaphoreType.DMA,
            pltpu.SemaphoreType.DMA,
        ],
        compiler_params=pltpu.CompilerParams(use_tc_tiling_on_sc=False),
    )


def _sc_agg(h_lo, h_hi, srcp, dstp):
    return _build_sc_agg()(h_lo, h_hi, srcp, dstp)


def _gin_body(lo_ref, hi_ref, wgin_ref, y_ref, sums_ref):
    i = pl.program_id(0)
    sfull = jnp.concatenate([lo_ref[...], hi_ref[...]], axis=1)
    y = lax.dot_general(sfull, wgin_ref[...],
                        (((1,), (1,)), ((), ())),
                        preferred_element_type=jnp.float32,
                        precision=lax.Precision.HIGHEST)
    y_ref[...] = y
    rows = i * BLK + lax.broadcasted_iota(jnp.int32, (BLK, 1), 0)
    ym = jnp.where(rows < N, y, 0.0)

    @pl.when(i == 0)
    def _():
        sums_ref[...] = jnp.zeros_like(sums_ref)

    sums_ref[0:1, :] += jnp.sum(ym, axis=0, keepdims=True)
    sums_ref[1:2, :] += jnp.sum(ym * ym, axis=0, keepdims=True)


def _gin(s_lo, s_hi, W_gin):
    return pl.pallas_call(
        _gin_body,
        grid=(GRID,),
        in_specs=[
            pl.BlockSpec((BLK, HALF), lambda i: (i, 0)),
            pl.BlockSpec((BLK, HALF), lambda i: (i, 0)),
            pl.BlockSpec((D_H, D_H), lambda i: (0, 0)),
        ],
        out_specs=[
            pl.BlockSpec((BLK, D_H), lambda i: (i, 0)),
            pl.BlockSpec((2, D_H), lambda i: (0, 0)),
        ],
        out_shape=[
            jax.ShapeDtypeStruct((NPAD, D_H), jnp.float32),
            jax.ShapeDtypeStruct((2, D_H), jnp.float32),
        ],
    )(s_lo, s_hi, W_gin)


def _read_body(y_ref, sums_ref, bng_ref, bnb_ref, wk_ref, bk_ref, wq_ref,
               wv_ref, bv_ref, wp0_ref, bp0_ref, wp1_ref, bp1_ref,
               w_ref, preds_ref, gh_ref):
    i = pl.program_id(0)
    mean = sums_ref[0:1, :] * np.float32(1.0 / N)
    ey2 = sums_ref[1:2, :] * np.float32(1.0 / N)
    inv = lax.rsqrt(ey2 - mean * mean + np.float32(1e-5))
    g = _gelu((y_ref[...] - mean) * inv * bng_ref[...] + bnb_ref[...])
    keys = lax.dot_general(g, wk_ref[...], (((1,), (1,)), ((), ())),
                           preferred_element_type=jnp.float32,
                           precision=lax.Precision.HIGHEST) + bk_ref[...]
    wgt = jax.nn.sigmoid(jnp.sum(keys * wq_ref[...], axis=1, keepdims=True))
    v = lax.dot_general(g, wv_ref[...], (((1,), (1,)), ((), ())),
                        preferred_element_type=jnp.float32,
                        precision=lax.Precision.HIGHEST) + bv_ref[...]
    wv = wgt * v
    rows = i * BLK + lax.broadcasted_iota(jnp.int32, (BLK, 1), 0)
    valid = rows < N
    seg0 = rows < N_GENE
    c0 = jnp.sum(jnp.where(valid & seg0, wv, 0.0), axis=0, keepdims=True)
    c1 = jnp.sum(jnp.where(valid & ~seg0, wv, 0.0), axis=0, keepdims=True)

    @pl.when(i == 0)
    def _():
        gh_ref[...] = jnp.zeros_like(gh_ref)
        preds_ref[...] = jnp.zeros_like(preds_ref)

    gh_ref[0:1, :] += c0
    gh_ref[1:2, :] += c1
    w_ref[...] = wgt

    @pl.when(i == GRID - 1)
    def _():
        gh = gh_ref[...]
        p = _gelu(lax.dot_general(gh, wp0_ref[...], (((1,), (1,)), ((), ())),
                                  preferred_element_type=jnp.float32,
                                  precision=lax.Precision.HIGHEST)
                  + bp0_ref[...])
        preds_ref[...] = (jnp.sum(p * wp1_ref[...], axis=1, keepdims=True)
                          + bp1_ref[...])


def _readout(y, sums, bn_g, bn_b, W_key, b_key, W_q, W_val, b_val,
             W_p0, b_p0, W_p1, b_p1):
    full = lambda r, c: pl.BlockSpec((r, c), lambda i: (0, 0))
    return pl.pallas_call(
        _read_body,
        grid=(GRID,),
        in_specs=[
            pl.BlockSpec((BLK, D_H), lambda i: (i, 0)),
            full(2, D_H),
            full(1, D_H), full(1, D_H),
            full(D_H, D_H), full(1, D_H), full(1, D_H),
            full(D_H, D_H), full(1, D_H),
            full(D_H, D_H), full(1, D_H),
            full(1, D_H), full(1, 1),
        ],
        out_specs=[
            pl.BlockSpec((BLK, 1), lambda i: (i, 0)),
            pl.BlockSpec((B, 1), lambda i: (0, 0)),
        ],
        out_shape=[
            jax.ShapeDtypeStruct((NPAD, 1), jnp.float32),
            jax.ShapeDtypeStruct((B, 1), jnp.float32),
        ],
        scratch_shapes=[pltpu.VMEM((B, D_H), jnp.float32)],
    )(y, sums, bn_g.reshape(1, D_H), bn_b.reshape(1, D_H),
      W_key, b_key.reshape(1, D_H), W_q,
      W_val, b_val.reshape(1, D_H),
      W_p0, b_p0.reshape(1, D_H), W_p1, b_p1.reshape(1, 1))


def kernel(x, edge_index, W_ge, b_ge, emb, W_gin, bn_g, bn_b,
           W_key, b_key, W_q, W_val, b_val, W_p0, b_p0, W_p1, b_p1):
    xf = jnp.pad(x.reshape(N, D_IN), ((0, NPAD - N), (0, 0)))
    emb2 = jnp.pad(jnp.concatenate([emb, emb], axis=0), ((0, NPAD - N), (0, 0)))
    src = jnp.concatenate([edge_index[0], jnp.zeros((E_PAD - E,), jnp.int32)])
    dst = jnp.concatenate([edge_index[1],
                           jnp.full((E_PAD - E,), N, jnp.int32)])
    srcp = src.reshape(16, NCHUNK, CHUNK)
    dstp = dst.reshape(16, NCHUNK, CHUNK)

    h_lo, h_hi = _encode(xf, emb2, W_ge, b_ge)
    s_lo, s_hi = _sc_agg(h_lo, h_hi, srcp, dstp)
    y, sums = _gin(s_lo, s_hi, W_gin)
    w_pad, preds = _readout(y, sums, bn_g, bn_b, W_key, b_key, W_q,
                            W_val, b_val, W_p0, b_p0, W_p1, b_p1)
    return preds, w_pad[:N]


# speedup vs baseline: 8.9042x; 1.1604x over previous
import functools

import jax
import jax.numpy as jnp
import numpy as np
from jax import lax
from jax.experimental import pallas as pl
from jax.experimental.pallas import tpu as pltpu
from jax.experimental.pallas import tpu_sc as plsc

B = 2
N_GENE = 19836
D_IN = 11
D_H = 64
N = B * N_GENE
E = 634752

NPAD = 39680
BLK = 2480
GRID = NPAD // BLK
ROWS_PER_TILE = NPAD // 16
CHUNK = 128
NCHUNK = 312
E_PAD = 16 * NCHUNK * CHUNK
HALF = D_H // 2

_SQRT1_2 = np.float32(0.7071067811865476)


def _gelu(t):
    return 0.5 * t * (1.0 + lax.erf(t * _SQRT1_2))


def _enc_body(x_ref, emb_ref, wge_ref, bge_ref, lo_ref, hi_ref):
    h = lax.dot_general(x_ref[...], wge_ref[...],
                        (((1,), (1,)), ((), ())),
                        preferred_element_type=jnp.float32,
                        precision=lax.Precision.HIGHEST)
    h = _gelu(h + bge_ref[...]) + emb_ref[...]
    lo_ref[...] = h[:, :HALF]
    hi_ref[...] = h[:, HALF:]


def _encode(xf, emb2, W_ge, b_ge):
    return pl.pallas_call(
        _enc_body,
        grid=(GRID,),
        in_specs=[
            pl.BlockSpec((BLK, D_IN), lambda i: (i, 0)),
            pl.BlockSpec((BLK, D_H), lambda i: (i, 0)),
            pl.BlockSpec((D_H, D_IN), lambda i: (0, 0)),
            pl.BlockSpec((1, D_H), lambda i: (0, 0)),
        ],
        out_specs=[
            pl.BlockSpec((BLK, HALF), lambda i: (i, 0)),
            pl.BlockSpec((BLK, HALF), lambda i: (i, 0)),
        ],
        out_shape=[
            jax.ShapeDtypeStruct((NPAD, HALF), jnp.float32),
            jax.ShapeDtypeStruct((NPAD, HALF), jnp.float32),
        ],
    )(xf, emb2, W_ge, b_ge.reshape(1, D_H))


STAGE = 52
NSTAGE = NCHUNK // STAGE
GROUPS = STAGE // 4


def _sc_body(hlo, hhi, ep, olo, ohi,
             acc, idx0, idx1, gbufs, gsems, isem0, isem1):
    c = lax.axis_index("c")
    s = lax.axis_index("s")

    def stage_load(t, idx, isem):
        pltpu.async_copy(ep.at[0, s, pl.ds(t * STAGE, STAGE)], idx.at[0], isem)
        pltpu.async_copy(ep.at[1, s, pl.ds(t * STAGE, STAGE)], idx.at[1], isem)

    def stage_wait(idx, isem):
        pltpu.make_async_copy(ep.at[0, 0, pl.ds(0, STAGE)], idx.at[0],
                              isem).wait()
        pltpu.make_async_copy(ep.at[0, 0, pl.ds(0, STAGE)], idx.at[1],
                              isem).wait()

    def run(h_hbm, out_hbm):
        rows = pl.ds(s * ROWS_PER_TILE, ROWS_PER_TILE)
        stage_load(0, idx0, isem0)
        pltpu.sync_copy(h_hbm.at[rows], acc.at[rows])
        plsc.subcore_barrier()

        for t in range(NSTAGE):
            idx, isem = (idx0, isem0) if t % 2 == 0 else (idx1, isem1)
            nidx, nisem = (idx1, isem1) if t % 2 == 0 else (idx0, isem0)
            stage_wait(idx, isem)
            if t + 1 < NSTAGE:
                stage_load(t + 1, nidx, nisem)
            for j in range(4):
                pltpu.async_copy(h_hbm.at[idx.at[0, j]], gbufs.at[j],
                                 gsems.at[j])

            def group(k, carry):
                base = k * 4
                for j in range(4):
                    pltpu.make_async_copy(h_hbm.at[pl.ds(0, CHUNK)],
                                          gbufs.at[j], gsems.at[j]).wait()
                    pltpu.sync_copy(gbufs.at[j], acc.at[idx.at[1, base + j]],
                                    add=True)

                    @pl.when(k + 1 < GROUPS)
                    def _():
                        pltpu.async_copy(h_hbm.at[idx.at[0, base + 4 + j]],
                                         gbufs.at[j], gsems.at[j])
                return carry

            lax.fori_loop(0, GROUPS, group, 0)
        plsc.subcore_barrier()
        pltpu.sync_copy(acc.at[rows], out_hbm.at[rows])

    @pl.when(c == 0)
    def _():
        run(hlo, olo)

    @pl.when(c == 1)
    def _():
        run(hhi, ohi)


@functools.cache
def _build_sc_agg():
    return pl.kernel(
        _sc_body,
        out_type=(
            jax.ShapeDtypeStruct((NPAD, HALF), jnp.float32),
            jax.ShapeDtypeStruct((NPAD, HALF), jnp.float32),
        ),
        mesh=plsc.VectorSubcoreMesh(core_axis_name="c", subcore_axis_name="s"),
        scratch_types=[
            pltpu.VMEM_SHARED((NPAD, HALF), jnp.float32),
            pltpu.VMEM((2, STAGE, CHUNK), jnp.int32),
            pltpu.VMEM((2, STAGE, CHUNK), jnp.int32),
            pltpu.VMEM((4, CHUNK, HALF), jnp.float32),
            pltpu.SemaphoreType.DMA((4,)),
            pltpu.SemaphoreType.DMA,
            pltpu.SemaphoreType.DMA,
        ],
        compiler_params=pltpu.CompilerParams(use_tc_tiling_on_sc=False),
    )


def _sc_agg(h_lo, h_hi, ep):
    return _build_sc_agg()(h_lo, h_hi, ep)


def _gin_body(lo_ref, hi_ref, wgin_ref, y_ref, sums_ref):
    i = pl.program_id(0)
    sfull = jnp.concatenate([lo_ref[...], hi_ref[...]], axis=1)
    y = lax.dot_general(sfull, wgin_ref[...],
                        (((1,), (1,)), ((), ())),
                        preferred_element_type=jnp.float32,
                        precision=lax.Precision.HIGHEST)
    y_ref[...] = y
    rows = i * BLK + lax.broadcasted_iota(jnp.int32, (BLK, 1), 0)
    ym = jnp.where(rows < N, y, 0.0)

    @pl.when(i == 0)
    def _():
        sums_ref[...] = jnp.zeros_like(sums_ref)

    sums_ref[0:1, :] += jnp.sum(ym, axis=0, keepdims=True)
    sums_ref[1:2, :] += jnp.sum(ym * ym, axis=0, keepdims=True)


def _gin(s_lo, s_hi, W_gin):
    return pl.pallas_call(
        _gin_body,
        grid=(GRID,),
        in_specs=[
            pl.BlockSpec((BLK, HALF), lambda i: (i, 0)),
            pl.BlockSpec((BLK, HALF), lambda i: (i, 0)),
            pl.BlockSpec((D_H, D_H), lambda i: (0, 0)),
        ],
        out_specs=[
            pl.BlockSpec((BLK, D_H), lambda i: (i, 0)),
            pl.BlockSpec((2, D_H), lambda i: (0, 0)),
        ],
        out_shape=[
            jax.ShapeDtypeStruct((NPAD, D_H), jnp.float32),
            jax.ShapeDtypeStruct((2, D_H), jnp.float32),
        ],
    )(s_lo, s_hi, W_gin)


def _read_body(y_ref, sums_ref, bng_ref, bnb_ref, wk_ref, bk_ref, wq_ref,
               wv_ref, bv_ref, wp0_ref, bp0_ref, wp1_ref, bp1_ref,
               w_ref, preds_ref, gh_ref):
    i = pl.program_id(0)
    mean = sums_ref[0:1, :] * np.float32(1.0 / N)
    ey2 = sums_ref[1:2, :] * np.float32(1.0 / N)
    inv = lax.rsqrt(ey2 - mean * mean + np.float32(1e-5))
    g = _gelu((y_ref[...] - mean) * inv * bng_ref[...] + bnb_ref[...])
    keys = lax.dot_general(g, wk_ref[...], (((1,), (1,)), ((), ())),
                           preferred_element_type=jnp.float32,
                           precision=lax.Precision.HIGHEST) + bk_ref[...]
    wgt = jax.nn.sigmoid(jnp.sum(keys * wq_ref[...], axis=1, keepdims=True))
    v = lax.dot_general(g, wv_ref[...], (((1,), (1,)), ((), ())),
                        preferred_element_type=jnp.float32,
                        precision=lax.Precision.HIGHEST) + bv_ref[...]
    wv = wgt * v
    rows = i * BLK + lax.broadcasted_iota(jnp.int32, (BLK, 1), 0)
    valid = rows < N
    seg0 = rows < N_GENE
    c0 = jnp.sum(jnp.where(valid & seg0, wv, 0.0), axis=0, keepdims=True)
    c1 = jnp.sum(jnp.where(valid & ~seg0, wv, 0.0), axis=0, keepdims=True)

    @pl.when(i == 0)
    def _():
        gh_ref[...] = jnp.zeros_like(gh_ref)
        preds_ref[...] = jnp.zeros_like(preds_ref)

    gh_ref[0:1, :] += c0
    gh_ref[1:2, :] += c1
    w_ref[...] = wgt

    @pl.when(i == GRID - 1)
    def _():
        gh = gh_ref[...]
        p = _gelu(lax.dot_general(gh, wp0_ref[...], (((1,), (1,)), ((), ())),
                                  preferred_element_type=jnp.float32,
                                  precision=lax.Precision.HIGHEST)
                  + bp0_ref[...])
        preds_ref[...] = (jnp.sum(p * wp1_ref[...], axis=1, keepdims=True)
                          + bp1_ref[...])


def _readout(y, sums, bn_g, bn_b, W_key, b_key, W_q, W_val, b_val,
             W_p0, b_p0, W_p1, b_p1):
    full = lambda r, c: pl.BlockSpec((r, c), lambda i: (0, 0))
    return pl.pallas_call(
        _read_body,
        grid=(GRID,),
        in_specs=[
            pl.BlockSpec((BLK, D_H), lambda i: (i, 0)),
            full(2, D_H),
            full(1, D_H), full(1, D_H),
            full(D_H, D_H), full(1, D_H), full(1, D_H),
            full(D_H, D_H), full(1, D_H),
            full(D_H, D_H), full(1, D_H),
            full(1, D_H), full(1, 1),
        ],
        out_specs=[
            pl.BlockSpec((BLK, 1), lambda i: (i, 0)),
            pl.BlockSpec((B, 1), lambda i: (0, 0)),
        ],
        out_shape=[
            jax.ShapeDtypeStruct((N, 1), jnp.float32),
            jax.ShapeDtypeStruct((B, 1), jnp.float32),
        ],
        scratch_shapes=[pltpu.VMEM((B, D_H), jnp.float32)],
    )(y, sums, bn_g.reshape(1, D_H), bn_b.reshape(1, D_H),
      W_key, b_key.reshape(1, D_H), W_q,
      W_val, b_val.reshape(1, D_H),
      W_p0, b_p0.reshape(1, D_H), W_p1, b_p1.reshape(1, 1))


def kernel(x, edge_index, W_ge, b_ge, emb, W_gin, bn_g, bn_b,
           W_key, b_key, W_q, W_val, b_val, W_p0, b_p0, W_p1, b_p1):
    xf = x.reshape(N, D_IN)
    emb2 = jnp.concatenate([emb, emb], axis=0)
    ep = jnp.pad(edge_index, ((0, 0), (0, E_PAD - E)),
                 constant_values=N).reshape(2, 16, NCHUNK, CHUNK)

    h_lo, h_hi = _encode(xf, emb2, W_ge, b_ge)
    s_lo, s_hi = _sc_agg(h_lo, h_hi, ep)
    y, sums = _gin(s_lo, s_hi, W_gin)
    w, preds = _readout(y, sums, bn_g, bn_b, W_key, b_key, W_q,
                        W_val, b_val, W_p0, b_p0, W_p1, b_p1)
    return preds, w


# speedup vs baseline: 9.2809x; 1.0423x over previous
import functools

import jax
import jax.numpy as jnp
import numpy as np
from jax import lax
from jax.experimental import pallas as pl
from jax.experimental.pallas import tpu as pltpu
from jax.experimental.pallas import tpu_sc as plsc

B = 2
N_GENE = 19836
D_IN = 11
D_H = 64
N = B * N_GENE
E = 634752

NPAD = 39680
BLK = 2480
GRID = NPAD // BLK
ROWS_PER_TILE = NPAD // 16
CHUNK = 128
NCHUNK = 312
E_PAD = 16 * NCHUNK * CHUNK
HALF = D_H // 2

_SQRT1_2 = np.float32(0.7071067811865476)


def _gelu(t):
    return 0.5 * t * (1.0 + lax.erf(t * _SQRT1_2))


def _enc_body(x_ref, emb_ref, wge_ref, bge_ref, lo_ref, hi_ref):
    h = lax.dot_general(x_ref[...], wge_ref[...],
                        (((1,), (1,)), ((), ())),
                        preferred_element_type=jnp.float32,
                        precision=lax.Precision.HIGHEST)
    h = _gelu(h + bge_ref[...]) + emb_ref[...]
    lo_ref[...] = h[:, :HALF]
    hi_ref[...] = h[:, HALF:]


def _encode(xf, emb2, W_ge, b_ge):
    return pl.pallas_call(
        _enc_body,
        grid=(GRID,),
        in_specs=[
            pl.BlockSpec((BLK, D_IN), lambda i: (i, 0)),
            pl.BlockSpec((BLK, D_H), lambda i: (i, 0)),
            pl.BlockSpec((D_H, D_IN), lambda i: (0, 0)),
            pl.BlockSpec((1, D_H), lambda i: (0, 0)),
        ],
        out_specs=[
            pl.BlockSpec((BLK, HALF), lambda i: (i, 0)),
            pl.BlockSpec((BLK, HALF), lambda i: (i, 0)),
        ],
        out_shape=[
            jax.ShapeDtypeStruct((NPAD, HALF), jnp.float32),
            jax.ShapeDtypeStruct((NPAD, HALF), jnp.float32),
        ],
    )(xf, emb2, W_ge, b_ge.reshape(1, D_H))


STAGE = 24
NSTAGE = NCHUNK // STAGE
RING = 8


def _sc_body(hlo, hhi, ep, olo, ohi,
             acc, idx0, idx1, gbufs, gsems, ssems, isem0, isem1):
    c = lax.axis_index("c")
    s = lax.axis_index("s")

    def stage_load(t, idx, isem):
        pltpu.async_copy(ep.at[0, s, pl.ds(t * STAGE, STAGE)], idx.at[0], isem)
        pltpu.async_copy(ep.at[1, s, pl.ds(t * STAGE, STAGE)], idx.at[1], isem)

    def stage_wait(idx, isem):
        pltpu.make_async_copy(ep.at[0, 0, pl.ds(0, STAGE)], idx.at[0],
                              isem).wait()
        pltpu.make_async_copy(ep.at[0, 0, pl.ds(0, STAGE)], idx.at[1],
                              isem).wait()

    def scatter_wait(j):
        pltpu.make_async_copy(gbufs.at[j], acc.at[pl.ds(0, CHUNK)],
                              ssems.at[j]).wait()

    def run(h_hbm, out_hbm):
        rows = pl.ds(s * ROWS_PER_TILE, ROWS_PER_TILE)
        stage_load(0, idx0, isem0)
        pltpu.sync_copy(h_hbm.at[rows], acc.at[rows])
        plsc.subcore_barrier()

        for t in range(NSTAGE):
            idx, isem = (idx0, isem0) if t % 2 == 0 else (idx1, isem1)
            nidx, nisem = (idx1, isem1) if t % 2 == 0 else (idx0, isem0)
            stage_wait(idx, isem)
            if t + 1 < NSTAGE:
                stage_load(t + 1, nidx, nisem)
            for j in range(RING):
                if t > 0:
                    scatter_wait(j)
                pltpu.async_copy(h_hbm.at[idx.at[0, j]], gbufs.at[j],
                                 gsems.at[j])

            def block(m, carry):
                base = m * RING
                for j in range(RING):
                    pltpu.make_async_copy(h_hbm.at[pl.ds(0, CHUNK)],
                                          gbufs.at[j], gsems.at[j]).wait()
                    pltpu.async_copy(gbufs.at[j], acc.at[idx.at[1, base + j]],
                                     ssems.at[j], add=True)

                    @pl.when(m + 1 < STAGE // RING)
                    def _():
                        scatter_wait(j)
                        pltpu.async_copy(h_hbm.at[idx.at[0, base + RING + j]],
                                         gbufs.at[j], gsems.at[j])
                return carry

            lax.fori_loop(0, STAGE // RING, block, 0)
        for j in range(RING):
            scatter_wait(j)
        plsc.subcore_barrier()
        pltpu.sync_copy(acc.at[rows], out_hbm.at[rows])

    @pl.when(c == 0)
    def _():
        run(hlo, olo)

    @pl.when(c == 1)
    def _():
        run(hhi, ohi)


@functools.cache
def _build_sc_agg():
    return pl.kernel(
        _sc_body,
        out_type=(
            jax.ShapeDtypeStruct((NPAD, HALF), jnp.float32),
            jax.ShapeDtypeStruct((NPAD, HALF), jnp.float32),
        ),
        mesh=plsc.VectorSubcoreMesh(core_axis_name="c", subcore_axis_name="s"),
        scratch_types=[
            pltpu.VMEM_SHARED((NPAD, HALF), jnp.float32),
            pltpu.VMEM((2, STAGE, CHUNK), jnp.int32),
            pltpu.VMEM((2, STAGE, CHUNK), jnp.int32),
            pltpu.VMEM((RING, CHUNK, HALF), jnp.float32),
            pltpu.SemaphoreType.DMA((RING,)),
            pltpu.SemaphoreType.DMA((RING,)),
            pltpu.SemaphoreType.DMA,
            pltpu.SemaphoreType.DMA,
        ],
        compiler_params=pltpu.CompilerParams(use_tc_tiling_on_sc=False),
    )


def _sc_agg(h_lo, h_hi, ep):
    return _build_sc_agg()(h_lo, h_hi, ep)


def _gin_body(lo_ref, hi_ref, wgin_ref, y_ref, sums_ref):
    i = pl.program_id(0)
    sfull = jnp.concatenate([lo_ref[...], hi_ref[...]], axis=1)
    y = lax.dot_general(sfull, wgin_ref[...],
                        (((1,), (1,)), ((), ())),
                        preferred_element_type=jnp.float32,
                        precision=lax.Precision.HIGHEST)
    y_ref[...] = y
    rows = i * BLK + lax.broadcasted_iota(jnp.int32, (BLK, 1), 0)
    ym = jnp.where(rows < N, y, 0.0)

    @pl.when(i == 0)
    def _():
        sums_ref[...] = jnp.zeros_like(sums_ref)

    sums_ref[0:1, :] += jnp.sum(ym, axis=0, keepdims=True)
    sums_ref[1:2, :] += jnp.sum(ym * ym, axis=0, keepdims=True)


def _gin(s_lo, s_hi, W_gin):
    return pl.pallas_call(
        _gin_body,
        grid=(GRID,),
        in_specs=[
            pl.BlockSpec((BLK, HALF), lambda i: (i, 0)),
            pl.BlockSpec((BLK, HALF), lambda i: (i, 0)),
            pl.BlockSpec((D_H, D_H), lambda i: (0, 0)),
        ],
        out_specs=[
            pl.BlockSpec((BLK, D_H), lambda i: (i, 0)),
            pl.BlockSpec((2, D_H), lambda i: (0, 0)),
        ],
        out_shape=[
            jax.ShapeDtypeStruct((NPAD, D_H), jnp.float32),
            jax.ShapeDtypeStruct((2, D_H), jnp.float32),
        ],
    )(s_lo, s_hi, W_gin)


def _read_body(y_ref, sums_ref, bng_ref, bnb_ref, wk_ref, bk_ref, wq_ref,
               wv_ref, bv_ref, wp0_ref, bp0_ref, wp1_ref, bp1_ref,
               w_ref, preds_ref, gh_ref):
    i = pl.program_id(0)
    mean = sums_ref[0:1, :] * np.float32(1.0 / N)
    ey2 = sums_ref[1:2, :] * np.float32(1.0 / N)
    inv = lax.rsqrt(ey2 - mean * mean + np.float32(1e-5))
    scale = inv * bng_ref[...]
    shift = bnb_ref[...] - mean * scale
    g = _gelu(y_ref[...] * scale + shift)
    qrow = lax.dot_general(wq_ref[...], wk_ref[...], (((1,), (0,)), ((), ())),
                           preferred_element_type=jnp.float32,
                           precision=lax.Precision.HIGHEST)
    qb = jnp.sum(bk_ref[...] * wq_ref[...], axis=1, keepdims=True)
    wgt = jax.nn.sigmoid(jnp.sum(g * qrow, axis=1, keepdims=True) + qb)
    v = lax.dot_general(g, wv_ref[...], (((1,), (1,)), ((), ())),
                        preferred_element_type=jnp.float32,
                        precision=lax.Precision.HIGHEST) + bv_ref[...]
    wv = wgt * v
    rows = i * BLK + lax.broadcasted_iota(jnp.int32, (BLK, 1), 0)
    valid = rows < N
    seg0 = rows < N_GENE
    c0 = jnp.sum(jnp.where(valid & seg0, wv, 0.0), axis=0, keepdims=True)
    c1 = jnp.sum(jnp.where(valid & ~seg0, wv, 0.0), axis=0, keepdims=True)

    @pl.when(i == 0)
    def _():
        gh_ref[...] = jnp.zeros_like(gh_ref)
        preds_ref[...] = jnp.zeros_like(preds_ref)

    gh_ref[0:1, :] += c0
    gh_ref[1:2, :] += c1
    w_ref[...] = wgt

    @pl.when(i == GRID - 1)
    def _():
        gh = gh_ref[...]
        p = _gelu(lax.dot_general(gh, wp0_ref[...], (((1,), (1,)), ((), ())),
                                  preferred_element_type=jnp.float32,
                                  precision=lax.Precision.HIGHEST)
                  + bp0_ref[...])
        preds_ref[...] = (jnp.sum(p * wp1_ref[...], axis=1, keepdims=True)
                          + bp1_ref[...])


def _readout(y, sums, bn_g, bn_b, W_key, b_key, W_q, W_val, b_val,
             W_p0, b_p0, W_p1, b_p1):
    full = lambda r, c: pl.BlockSpec((r, c), lambda i: (0, 0))
    return pl.pallas_call(
        _read_body,
        grid=(GRID,),
        in_specs=[
            pl.BlockSpec((BLK, D_H), lambda i: (i, 0)),
            full(2, D_H),
            full(1, D_H), full(1, D_H),
            full(D_H, D_H), full(1, D_H), full(1, D_H),
            full(D_H, D_H), full(1, D_H),
            full(D_H, D_H), full(1, D_H),
            full(1, D_H), full(1, 1),
        ],
        out_specs=[
            pl.BlockSpec((BLK, 1), lambda i: (i, 0)),
            pl.BlockSpec((B, 1), lambda i: (0, 0)),
        ],
        out_shape=[
            jax.ShapeDtypeStruct((N, 1), jnp.float32),
            jax.ShapeDtypeStruct((B, 1), jnp.float32),
        ],
        scratch_shapes=[pltpu.VMEM((B, D_H), jnp.float32)],
    )(y, sums, bn_g.reshape(1, D_H), bn_b.reshape(1, D_H),
      W_key, b_key.reshape(1, D_H), W_q,
      W_val, b_val.reshape(1, D_H),
      W_p0, b_p0.reshape(1, D_H), W_p1, b_p1.reshape(1, 1))


def kernel(x, edge_index, W_ge, b_ge, emb, W_gin, bn_g, bn_b,
           W_key, b_key, W_q, W_val, b_val, W_p0, b_p0, W_p1, b_p1):
    xf = x.reshape(N, D_IN)
    emb2 = jnp.concatenate([emb, emb], axis=0)
    ep = jnp.pad(edge_index, ((0, 0), (0, E_PAD - E)),
                 constant_values=N).reshape(2, 16, NCHUNK, CHUNK)

    h_lo, h_hi = _encode(xf, emb2, W_ge, b_ge)
    s_lo, s_hi = _sc_agg(h_lo, h_hi, ep)
    y, sums = _gin(s_lo, s_hi, W_gin)
    w, preds = _readout(y, sums, bn_g, bn_b, W_key, b_key, W_q,
                        W_val, b_val, W_p0, b_p0, W_p1, b_p1)
    return preds, w


# speedup vs baseline: 9.9187x; 1.0687x over previous
import functools

import jax
import jax.numpy as jnp
import numpy as np
from jax import lax
from jax.experimental import pallas as pl
from jax.experimental.pallas import tpu as pltpu
from jax.experimental.pallas import tpu_sc as plsc

B = 2
N_GENE = 19836
D_IN = 11
D_H = 64
N = B * N_GENE
E = 634752

NPAD = 39680
BLK = 2480
GRID = NPAD // BLK
ROWS_PER_TILE = NPAD // 16
CHUNK = 128
NCHUNK = 312
E_PAD = 16 * NCHUNK * CHUNK
HALF = D_H // 2

_SQRT1_2 = np.float32(0.7071067811865476)


def _gelu(t):
    return 0.5 * t * (1.0 + lax.erf(t * _SQRT1_2))


def _enc_body(x_ref, emb_ref, wge_ref, bge_ref, lo_ref, hi_ref):
    h = lax.dot_general(x_ref[...], wge_ref[...],
                        (((1,), (1,)), ((), ())),
                        preferred_element_type=jnp.float32)
    h = _gelu(h + bge_ref[...]) + emb_ref[...]
    lo_ref[...] = h[:, :HALF]
    hi_ref[...] = h[:, HALF:]


def _encode(xf, emb2, W_ge, b_ge):
    return pl.pallas_call(
        _enc_body,
        grid=(GRID,),
        in_specs=[
            pl.BlockSpec((BLK, D_IN), lambda i: (i, 0)),
            pl.BlockSpec((BLK, D_H), lambda i: (i, 0)),
            pl.BlockSpec((D_H, D_IN), lambda i: (0, 0)),
            pl.BlockSpec((1, D_H), lambda i: (0, 0)),
        ],
        out_specs=[
            pl.BlockSpec((BLK, HALF), lambda i: (i, 0)),
            pl.BlockSpec((BLK, HALF), lambda i: (i, 0)),
        ],
        out_shape=[
            jax.ShapeDtypeStruct((NPAD, HALF), jnp.float32),
            jax.ShapeDtypeStruct((NPAD, HALF), jnp.float32),
        ],
    )(xf, emb2, W_ge, b_ge.reshape(1, D_H))


STAGE = 24
NSTAGE = NCHUNK // STAGE
RING = 8


def _sc_body(hlo, hhi, ep, olo, ohi,
             acc, idx0, idx1, gbufs, gsems, ssems, isem0, isem1):
    c = lax.axis_index("c")
    s = lax.axis_index("s")

    def stage_load(t, idx, isem):
        pltpu.async_copy(ep.at[0, s, pl.ds(t * STAGE, STAGE)], idx.at[0], isem)
        pltpu.async_copy(ep.at[1, s, pl.ds(t * STAGE, STAGE)], idx.at[1], isem)

    def stage_wait(idx, isem):
        pltpu.make_async_copy(ep.at[0, 0, pl.ds(0, STAGE)], idx.at[0],
                              isem).wait()
        pltpu.make_async_copy(ep.at[0, 0, pl.ds(0, STAGE)], idx.at[1],
                              isem).wait()

    def scatter_wait(j):
        pltpu.make_async_copy(gbufs.at[j], acc.at[pl.ds(0, CHUNK)],
                              ssems.at[j]).wait()

    def run(h_hbm, out_hbm):
        rows = pl.ds(s * ROWS_PER_TILE, ROWS_PER_TILE)
        stage_load(0, idx0, isem0)
        pltpu.sync_copy(h_hbm.at[rows], acc.at[rows])
        plsc.subcore_barrier()

        for t in range(NSTAGE):
            idx, isem = (idx0, isem0) if t % 2 == 0 else (idx1, isem1)
            nidx, nisem = (idx1, isem1) if t % 2 == 0 else (idx0, isem0)
            stage_wait(idx, isem)
            if t + 1 < NSTAGE:
                stage_load(t + 1, nidx, nisem)
            for j in range(RING):
                if t > 0:
                    scatter_wait(j)
                pltpu.async_copy(h_hbm.at[idx.at[0, j]], gbufs.at[j],
                                 gsems.at[j])

            def block(m, carry):
                base = m * RING
                for j in range(RING):
                    pltpu.make_async_copy(h_hbm.at[pl.ds(0, CHUNK)],
                                          gbufs.at[j], gsems.at[j]).wait()
                    pltpu.async_copy(gbufs.at[j], acc.at[idx.at[1, base + j]],
                                     ssems.at[j], add=True)

                    @pl.when(m + 1 < STAGE // RING)
                    def _():
                        scatter_wait(j)
                        pltpu.async_copy(h_hbm.at[idx.at[0, base + RING + j]],
                                         gbufs.at[j], gsems.at[j])
                return carry

            lax.fori_loop(0, STAGE // RING, block, 0)
        for j in range(RING):
            scatter_wait(j)
        plsc.subcore_barrier()
        pltpu.sync_copy(acc.at[rows], out_hbm.at[rows])

    @pl.when(c == 0)
    def _():
        run(hlo, olo)

    @pl.when(c == 1)
    def _():
        run(hhi, ohi)


@functools.cache
def _build_sc_agg():
    return pl.kernel(
        _sc_body,
        out_type=(
            jax.ShapeDtypeStruct((NPAD, HALF), jnp.float32),
            jax.ShapeDtypeStruct((NPAD, HALF), jnp.float32),
        ),
        mesh=plsc.VectorSubcoreMesh(core_axis_name="c", subcore_axis_name="s"),
        scratch_types=[
            pltpu.VMEM_SHARED((NPAD, HALF), jnp.float32),
            pltpu.VMEM((2, STAGE, CHUNK), jnp.int32),
            pltpu.VMEM((2, STAGE, CHUNK), jnp.int32),
            pltpu.VMEM((RING, CHUNK, HALF), jnp.float32),
            pltpu.SemaphoreType.DMA((RING,)),
            pltpu.SemaphoreType.DMA((RING,)),
            pltpu.SemaphoreType.DMA,
            pltpu.SemaphoreType.DMA,
        ],
        compiler_params=pltpu.CompilerParams(use_tc_tiling_on_sc=False),
    )


def _sc_agg(h_lo, h_hi, ep):
    return _build_sc_agg()(h_lo, h_hi, ep)


def _gin_body(lo_ref, hi_ref, wgin_ref, y_ref, sums_ref):
    i = pl.program_id(0)
    sfull = jnp.concatenate([lo_ref[...], hi_ref[...]], axis=1)
    y = lax.dot_general(sfull, wgin_ref[...],
                        (((1,), (1,)), ((), ())),
                        preferred_element_type=jnp.float32)
    y_ref[...] = y
    rows = i * BLK + lax.broadcasted_iota(jnp.int32, (BLK, 1), 0)
    ym = jnp.where(rows < N, y, 0.0)

    @pl.when(i == 0)
    def _():
        sums_ref[...] = jnp.zeros_like(sums_ref)

    sums_ref[0:1, :] += jnp.sum(ym, axis=0, keepdims=True)
    sums_ref[1:2, :] += jnp.sum(ym * ym, axis=0, keepdims=True)


def _gin(s_lo, s_hi, W_gin):
    return pl.pallas_call(
        _gin_body,
        grid=(GRID,),
        in_specs=[
            pl.BlockSpec((BLK, HALF), lambda i: (i, 0)),
            pl.BlockSpec((BLK, HALF), lambda i: (i, 0)),
            pl.BlockSpec((D_H, D_H), lambda i: (0, 0)),
        ],
        out_specs=[
            pl.BlockSpec((BLK, D_H), lambda i: (i, 0)),
            pl.BlockSpec((2, D_H), lambda i: (0, 0)),
        ],
        out_shape=[
            jax.ShapeDtypeStruct((NPAD, D_H), jnp.float32),
            jax.ShapeDtypeStruct((2, D_H), jnp.float32),
        ],
    )(s_lo, s_hi, W_gin)


def _read_body(y_ref, sums_ref, bng_ref, bnb_ref, wk_ref, bk_ref, wq_ref,
               wqt_ref, wv_ref, bv_ref, wp0_ref, bp0t_ref, wp1t_ref, bp1_ref,
               w_ref, preds_ref, gh_ref):
    i = pl.program_id(0)
    mean = sums_ref[0:1, :] * np.float32(1.0 / N)
    ey2 = sums_ref[1:2, :] * np.float32(1.0 / N)
    inv = 1.0 / jnp.sqrt(ey2 - mean * mean + np.float32(1e-5))
    scale = inv * bng_ref[...]
    shift = bnb_ref[...] - mean * scale
    g = _gelu(y_ref[...] * scale + shift)
    qrow = jnp.sum(wk_ref[...] * wqt_ref[...], axis=0, keepdims=True)
    qb = jnp.sum(bk_ref[...] * wq_ref[...], axis=1, keepdims=True)
    wgt = jax.nn.sigmoid(jnp.sum(g * qrow, axis=1, keepdims=True) + qb)
    v = lax.dot_general(g, wv_ref[...], (((1,), (1,)), ((), ())),
                        preferred_element_type=jnp.float32) + bv_ref[...]
    wv = wgt * v
    rows = i * BLK + lax.broadcasted_iota(jnp.int32, (BLK, 1), 0)
    valid = rows < N
    seg0 = rows < N_GENE
    c0 = jnp.sum(jnp.where(valid & seg0, wv, 0.0), axis=0, keepdims=True)
    c1 = jnp.sum(jnp.where(valid & ~seg0, wv, 0.0), axis=0, keepdims=True)

    @pl.when(i == 0)
    def _():
        gh_ref[...] = jnp.zeros_like(gh_ref)
        preds_ref[...] = jnp.zeros_like(preds_ref)

    gh_ref[0:1, :] += c0
    gh_ref[1:2, :] += c1
    w_ref[...] = wgt

    @pl.when(i == GRID - 1)
    def _():
        bf = lambda t: t.astype(jnp.bfloat16).astype(jnp.float32)
        wp0b = bf(wp0_ref[...])
        wp1b = bf(wp1t_ref[...])
        for b in range(B):
            ghb = bf(gh_ref[b:b + 1, :])
            z = jnp.sum(wp0b * ghb, axis=1, keepdims=True)
            p = bf(_gelu(z + bp0t_ref[...]))
            preds_ref[b:b + 1, :] = (
                jnp.sum(p * wp1b, axis=0, keepdims=True)
                + bp1_ref[...])


def _readout(y, sums, bn_g, bn_b, W_key, b_key, W_q, W_val, b_val,
             W_p0, b_p0, W_p1, b_p1):
    full = lambda r, c: pl.BlockSpec((r, c), lambda i: (0, 0))
    return pl.pallas_call(
        _read_body,
        grid=(GRID,),
        in_specs=[
            pl.BlockSpec((BLK, D_H), lambda i: (i, 0)),
            full(2, D_H),
            full(1, D_H), full(1, D_H),
            full(D_H, D_H), full(1, D_H), full(1, D_H), full(D_H, 1),
            full(D_H, D_H), full(1, D_H),
            full(D_H, D_H), full(D_H, 1),
            full(D_H, 1), full(1, 1),
        ],
        out_specs=[
            pl.BlockSpec((BLK, 1), lambda i: (i, 0)),
            pl.BlockSpec((B, 1), lambda i: (0, 0)),
        ],
        out_shape=[
            jax.ShapeDtypeStruct((N, 1), jnp.float32),
            jax.ShapeDtypeStruct((B, 1), jnp.float32),
        ],
        scratch_shapes=[pltpu.VMEM((B, D_H), jnp.float32)],
    )(y, sums, bn_g.reshape(1, D_H), bn_b.reshape(1, D_H),
      W_key, b_key.reshape(1, D_H), W_q, W_q.reshape(D_H, 1),
      W_val, b_val.reshape(1, D_H),
      W_p0, b_p0.reshape(D_H, 1), W_p1.reshape(D_H, 1),
      b_p1.reshape(1, 1))


def kernel(x, edge_index, W_ge, b_ge, emb, W_gin, bn_g, bn_b,
           W_key, b_key, W_q, W_val, b_val, W_p0, b_p0, W_p1, b_p1):
    xf = x.reshape(N, D_IN)
    emb2 = jnp.concatenate([emb, emb], axis=0)
    ep = jnp.pad(edge_index, ((0, 0), (0, E_PAD - E)),
                 constant_values=N).reshape(2, 16, NCHUNK, CHUNK)

    h_lo, h_hi = _encode(xf, emb2, W_ge, b_ge)
    s_lo, s_hi = _sc_agg(h_lo, h_hi, ep)
    y, sums = _gin(s_lo, s_hi, W_gin)
    w, preds = _readout(y, sums, bn_g, bn_b, W_key, b_key, W_q,
                        W_val, b_val, W_p0, b_p0, W_p1, b_p1)
    return preds, w
